# baseline (speedup 1.0000x reference)
import jax
import jax.numpy as jnp
from jax.experimental import pallas as pl
from jax.experimental.pallas import tpu as pltpu

N = 10000
F = 128
TM = 400


def _body(feat_ref, w_ref, adj_ref, out_ref, xw_ref):
    i = pl.program_id(0)

    @pl.when(i == 0)
    def _():
        xw_ref[...] = jnp.dot(
            feat_ref[...], w_ref[...], preferred_element_type=jnp.float32
        ).astype(jnp.bfloat16)

    out_ref[...] = jnp.dot(
        adj_ref[...].astype(jnp.bfloat16),
        xw_ref[...],
        preferred_element_type=jnp.float32,
    )


def kernel(feat, adj, weight):
    return pl.pallas_call(
        _body,
        grid=(N // TM,),
        in_specs=[
            pl.BlockSpec((N, F), lambda i: (0, 0)),
            pl.BlockSpec((F, F), lambda i: (0, 0)),
            pl.BlockSpec((TM, N), lambda i: (i, 0)),
        ],
        out_specs=pl.BlockSpec((TM, F), lambda i: (i, 0)),
        out_shape=jax.ShapeDtypeStruct((N, F), jnp.float32),
        scratch_shapes=[pltpu.VMEM((N, F), jnp.bfloat16)],
    )(feat, weight, adj)


# speedup vs baseline: 1.0076x; 1.0076x over previous
import jax
import jax.numpy as jnp
from jax.experimental import pallas as pl
from jax.experimental.pallas import tpu as pltpu

N = 10000
F = 128
TM = 200


def _body(feat_ref, w_ref, adj_ref, out_ref, xw_ref):
    i = pl.program_id(0)

    @pl.when(i == 0)
    def _():
        xw_ref[...] = jnp.dot(
            feat_ref[...], w_ref[...], preferred_element_type=jnp.float32
        ).astype(jnp.bfloat16)

    out_ref[...] = jnp.dot(
        adj_ref[...].astype(jnp.bfloat16),
        xw_ref[...],
        preferred_element_type=jnp.float32,
    )


def kernel(feat, adj, weight):
    return pl.pallas_call(
        _body,
        grid=(N // TM,),
        in_specs=[
            pl.BlockSpec((N, F), lambda i: (0, 0)),
            pl.BlockSpec((F, F), lambda i: (0, 0)),
            pl.BlockSpec((TM, N), lambda i: (i, 0)),
        ],
        out_specs=pl.BlockSpec((TM, F), lambda i: (i, 0)),
        out_shape=jax.ShapeDtypeStruct((N, F), jnp.float32),
        scratch_shapes=[pltpu.VMEM((N, F), jnp.bfloat16)],
    )(feat, weight, adj)
